# Initial kernel scaffold; baseline (speedup 1.0000x reference)
import jax
import jax.numpy as jnp
from jax.experimental import pallas as pl


def kernel(x, edge_index, edge_attr, W1, b1, W2, b2, W3, b3, Wfc, bfc):
    raise NotImplementedError("write your pallas kernel here")


# speedup vs baseline: 12.0223x; 12.0223x over previous
import functools

import jax
import jax.numpy as jnp
from jax import lax
from jax.experimental import pallas as pl
from jax.experimental.pallas import tpu as pltpu
from jax.experimental.pallas import tpu_sc as plsc

_N = 10000
_E = 320000
_DIN = 128
_H = 64
_C = 15
_CP = 16
_NC = 2
_NS = 16
_NW = _NC * _NS
_EPT = _E // _NW
_K = 80
_NCH = _EPT // _K
_NP = 10240
_RPT = _NP // _NS


def _sc_deg_body(dsts_hbm, ones_hbm, zeros_hbm, out_hbm, dst_v, ones_v, acc):
    c = lax.axis_index("c")
    s = lax.axis_index("s")
    wid = s * _NC + c
    pltpu.sync_copy(dsts_hbm.at[wid], dst_v)
    pltpu.sync_copy(ones_hbm, ones_v)
    pltpu.sync_copy(zeros_hbm.at[pl.ds(s * _RPT, _RPT)],
                    acc.at[pl.ds(s * _RPT, _RPT)])
    plsc.subcore_barrier()

    def step(j, carry):
        pltpu.sync_copy(ones_v, acc.at[dst_v.at[j]], add=True)
        return carry

    lax.fori_loop(0, _NCH, step, 0)
    plsc.subcore_barrier()
    pltpu.sync_copy(acc.at[pl.ds(s * _RPT, _RPT)],
                    out_hbm.at[c, pl.ds(s * _RPT, _RPT)])


@functools.cache
def _mesh():
    return plsc.VectorSubcoreMesh(
        core_axis_name="c", subcore_axis_name="s",
        num_cores=_NC, num_subcores=_NS)


@functools.cache
def _sc_deg():
    return pl.kernel(
        _sc_deg_body,
        out_type=jax.ShapeDtypeStruct((_NC, _NP, _CP), jnp.float32),
        mesh=_mesh(),
        compiler_params=pltpu.CompilerParams(use_tc_tiling_on_sc=False),
        scratch_types=[
            pltpu.VMEM((_NCH, _K), jnp.int32),
            pltpu.VMEM((_K, _CP), jnp.float32),
            pltpu.VMEM_SHARED((_NP, _CP), jnp.float32),
        ],
    )


def _sc_scatter_body(g_hbm, srcs_hbm, dsts_hbm, zeros_hbm, out_hbm,
                     src_v, dst_v, bufa, bufb, acc, sema, semb):
    c = lax.axis_index("c")
    s = lax.axis_index("s")
    wid = s * _NC + c
    pltpu.sync_copy(srcs_hbm.at[wid], src_v)
    pltpu.sync_copy(dsts_hbm.at[wid], dst_v)
    pltpu.sync_copy(zeros_hbm.at[pl.ds(s * _RPT, _RPT)],
                    acc.at[pl.ds(s * _RPT, _RPT)])
    plsc.subcore_barrier()

    pltpu.async_copy(g_hbm.at[src_v.at[0]], bufa, sema)

    def step(i, carry):
        j = 2 * i
        pltpu.async_copy(g_hbm.at[src_v.at[j + 1]], bufb, semb)
        pltpu.make_async_copy(g_hbm.at[src_v.at[j]], bufa, sema).wait()
        pltpu.sync_copy(bufa, acc.at[dst_v.at[j]], add=True)
        pltpu.async_copy(g_hbm.at[src_v.at[j + 2]], bufa, sema)
        pltpu.make_async_copy(g_hbm.at[src_v.at[j + 1]], bufb, semb).wait()
        pltpu.sync_copy(bufb, acc.at[dst_v.at[j + 1]], add=True)
        return carry

    lax.fori_loop(0, (_NCH - 1) // 2, step, 0)
    pltpu.make_async_copy(g_hbm.at[src_v.at[_NCH - 1]], bufa, sema).wait()
    pltpu.sync_copy(bufa, acc.at[dst_v.at[_NCH - 1]], add=True)
    plsc.subcore_barrier()
    pltpu.sync_copy(acc.at[pl.ds(s * _RPT, _RPT)],
                    out_hbm.at[c, pl.ds(s * _RPT, _RPT)])


@functools.cache
def _sc_scatter():
    return pl.kernel(
        _sc_scatter_body,
        out_type=jax.ShapeDtypeStruct((_NC, _NP, _H), jnp.float32),
        mesh=_mesh(),
        compiler_params=pltpu.CompilerParams(use_tc_tiling_on_sc=False),
        scratch_types=[
            pltpu.VMEM((_NCH, _K), jnp.int32),
            pltpu.VMEM((_NCH, _K), jnp.int32),
            pltpu.VMEM((_K, _H), jnp.float32),
            pltpu.VMEM((_K, _H), jnp.float32),
            pltpu.VMEM_SHARED((_NP, _H), jnp.float32),
            pltpu.SemaphoreType.DMA,
            pltpu.SemaphoreType.DMA,
        ],
    )


def _sc_gather_body(p_hbm, q_hbm, srcs_hbm, dsts_hbm, o1_hbm, o2_hbm,
                    src_v, dst_v, bufp, bufq, semp, semq):
    c = lax.axis_index("c")
    s = lax.axis_index("s")
    wid = s * _NC + c
    base = wid * _EPT
    pltpu.sync_copy(srcs_hbm.at[wid], src_v)
    pltpu.sync_copy(dsts_hbm.at[wid], dst_v)

    def step(j, carry):
        pltpu.async_copy(p_hbm.at[src_v.at[j]], bufp, semp)
        pltpu.async_copy(q_hbm.at[dst_v.at[j]], bufq, semq)
        pltpu.make_async_copy(p_hbm.at[src_v.at[j]], bufp, semp).wait()
        pltpu.sync_copy(bufp, o1_hbm.at[pl.ds(base + j * _K, _K)])
        pltpu.make_async_copy(q_hbm.at[dst_v.at[j]], bufq, semq).wait()
        pltpu.sync_copy(bufq, o2_hbm.at[pl.ds(base + j * _K, _K)])
        return carry

    lax.fori_loop(0, _NCH, step, 0)


@functools.cache
def _sc_gather():
    return pl.kernel(
        _sc_gather_body,
        out_type=[
            jax.ShapeDtypeStruct((_E, _CP), jnp.float32),
            jax.ShapeDtypeStruct((_E, _CP), jnp.float32),
        ],
        mesh=_mesh(),
        compiler_params=pltpu.CompilerParams(use_tc_tiling_on_sc=False),
        scratch_types=[
            pltpu.VMEM((_NCH, _K), jnp.int32),
            pltpu.VMEM((_NCH, _K), jnp.int32),
            pltpu.VMEM((_K, _CP), jnp.float32),
            pltpu.VMEM((_K, _CP), jnp.float32),
            pltpu.SemaphoreType.DMA,
            pltpu.SemaphoreType.DMA,
        ],
    )


_RB = 2000
_EB = 4000


def _a1_body(deg_ref, x_ref, w_ref, g_ref, dinv_ref):
    deg = (deg_ref[0] + deg_ref[1])[:, :1] + 1.0
    dinv = lax.rsqrt(deg)
    g_ref[...] = jnp.dot(x_ref[...], w_ref[...],
                         preferred_element_type=jnp.float32) * dinv
    dinv_ref[...] = dinv


def _a1(deg, x, w):
    return pl.pallas_call(
        _a1_body,
        grid=(_N // _RB,),
        in_specs=[
            pl.BlockSpec((_NC, _RB, _CP), lambda i: (0, i, 0)),
            pl.BlockSpec((_RB, _DIN), lambda i: (i, 0)),
            pl.BlockSpec((_DIN, _H), lambda i: (0, 0)),
        ],
        out_specs=[
            pl.BlockSpec((_RB, _H), lambda i: (i, 0)),
            pl.BlockSpec((_RB, 1), lambda i: (i, 0)),
        ],
        out_shape=[
            jax.ShapeDtypeStruct((_N, _H), jnp.float32),
            jax.ShapeDtypeStruct((_N, 1), jnp.float32),
        ],
    )(deg, x, w)


def _a2_body(g_ref, agg_ref, dinv_ref, w_ref, b_ref, out_ref):
    dinv = dinv_ref[...]
    h = jnp.maximum(dinv * (g_ref[...] + agg_ref[0] + agg_ref[1]) + b_ref[...],
                    0.0)
    out_ref[...] = jnp.dot(h, w_ref[...],
                           preferred_element_type=jnp.float32) * dinv


def _a2(g, agg, dinv, w, b):
    return pl.pallas_call(
        _a2_body,
        grid=(_N // _RB,),
        in_specs=[
            pl.BlockSpec((_RB, _H), lambda i: (i, 0)),
            pl.BlockSpec((_NC, _RB, _H), lambda i: (0, i, 0)),
            pl.BlockSpec((_RB, 1), lambda i: (i, 0)),
            pl.BlockSpec((_H, _H), lambda i: (0, 0)),
            pl.BlockSpec((1, _H), lambda i: (0, 0)),
        ],
        out_specs=pl.BlockSpec((_RB, _H), lambda i: (i, 0)),
        out_shape=jax.ShapeDtypeStruct((_N, _H), jnp.float32),
    )(g, agg, dinv, w, b)


def _a4_body(g_ref, agg_ref, dinv_ref, b_ref, wpq_ref, p_ref, q_ref):
    dinv = dinv_ref[...]
    h = jnp.maximum(dinv * (g_ref[...] + agg_ref[0] + agg_ref[1]) + b_ref[...],
                    0.0)
    pq = jnp.dot(h, wpq_ref[...], preferred_element_type=jnp.float32)
    p_ref[...] = pq[:, :_CP]
    q_ref[...] = pq[:, _CP:]


def _a4(g, agg, dinv, b, wpq):
    return pl.pallas_call(
        _a4_body,
        grid=(_N // _RB,),
        in_specs=[
            pl.BlockSpec((_RB, _H), lambda i: (i, 0)),
            pl.BlockSpec((_NC, _RB, _H), lambda i: (0, i, 0)),
            pl.BlockSpec((_RB, 1), lambda i: (i, 0)),
            pl.BlockSpec((1, _H), lambda i: (0, 0)),
            pl.BlockSpec((_H, 2 * _CP), lambda i: (0, 0)),
        ],
        out_specs=[
            pl.BlockSpec((_RB, _CP), lambda i: (i, 0)),
            pl.BlockSpec((_RB, _CP), lambda i: (i, 0)),
        ],
        out_shape=[
            jax.ShapeDtypeStruct((_N, _CP), jnp.float32),
            jax.ShapeDtypeStruct((_N, _CP), jnp.float32),
        ],
    )(g, agg, dinv, b, wpq)


def _a5_body(o1_ref, o2_ref, ea_ref, we_ref, b_ref, out_ref):
    ea = ea_ref[...]
    z = o1_ref[...] + o2_ref[...] + b_ref[...]
    for k in range(4):
        z = z + ea[:, k:k + 1] * we_ref[k:k + 1, :]
    m = jnp.max(z, axis=1, keepdims=True)
    lse = jnp.log(jnp.sum(jnp.exp(z - m), axis=1, keepdims=True))
    out_ref[...] = (z - m - lse)[:, :_C]


def _a5(o1, o2, ea, we, b):
    return pl.pallas_call(
        _a5_body,
        grid=(_E // _EB,),
        in_specs=[
            pl.BlockSpec((_EB, _CP), lambda i: (i, 0)),
            pl.BlockSpec((_EB, _CP), lambda i: (i, 0)),
            pl.BlockSpec((_EB, 4), lambda i: (i, 0)),
            pl.BlockSpec((4, _CP), lambda i: (0, 0)),
            pl.BlockSpec((1, _CP), lambda i: (0, 0)),
        ],
        out_specs=pl.BlockSpec((_EB, _C), lambda i: (i, 0)),
        out_shape=jax.ShapeDtypeStruct((_E, _C), jnp.float32),
    )(o1, o2, ea, we, b)


def kernel(x, edge_index, edge_attr, W1, b1, W2, b2, W3, b3, Wfc, bfc):
    src = edge_index[0].reshape(_NW, _NCH, _K)
    dst = edge_index[1].reshape(_NW, _NCH, _K)
    zeros_h = jnp.zeros((_NP, _H), jnp.float32)
    zeros_c = jnp.zeros((_NP, _CP), jnp.float32)
    ones_c = jnp.ones((_K, _CP), jnp.float32)

    wp = jnp.pad(Wfc[:_H], ((0, 0), (0, _CP - _C)))
    wq = jnp.pad(Wfc[_H:2 * _H], ((0, 0), (0, _CP - _C)))
    wpq = jnp.concatenate([wp, wq], axis=1)
    we = jnp.pad(Wfc[2 * _H:], ((0, 0), (0, _CP - _C)))
    bpad = jnp.concatenate(
        [bfc, jnp.full((_CP - _C,), -1e30, jnp.float32)]).reshape(1, _CP)

    deg = _sc_deg()(dst, ones_c, zeros_c)
    g1, dinv = _a1(deg, x, W1)
    agg1 = _sc_scatter()(g1, src, dst, zeros_h)
    g2 = _a2(g1, agg1, dinv, W2, b1.reshape(1, _H))
    agg2 = _sc_scatter()(g2, src, dst, zeros_h)
    g3 = _a2(g2, agg2, dinv, W3, b2.reshape(1, _H))
    agg3 = _sc_scatter()(g3, src, dst, zeros_h)
    p, q = _a4(g3, agg3, dinv, b3.reshape(1, _H), wpq)
    o1, o2 = _sc_gather()(p, q, src, dst)
    return _a5(o1, o2, edge_attr, we, bpad)


# speedup vs baseline: 17.1891x; 1.4298x over previous
import functools

import jax
import jax.numpy as jnp
from jax import lax
from jax.experimental import pallas as pl
from jax.experimental.pallas import tpu as pltpu
from jax.experimental.pallas import tpu_sc as plsc

_N = 10000
_E = 320000
_DIN = 128
_H = 64
_C = 15
_CP = 16
_NC = 2
_NS = 16
_NW = _NC * _NS
_EPT = _E // _NW
_K = 80
_NCH = _EPT // _K
_NP = 10240
_RPT = _NP // _NS


@functools.cache
def _mesh():
    return plsc.VectorSubcoreMesh(
        core_axis_name="c", subcore_axis_name="s",
        num_cores=_NC, num_subcores=_NS)


_SC_PARAMS = pltpu.CompilerParams(use_tc_tiling_on_sc=False)


def _sc_deg_body(dsts_hbm, ones_hbm, zeros_hbm, out_hbm, dst_v, ones_v, acc):
    c = lax.axis_index("c")
    s = lax.axis_index("s")
    wid = s * _NC + c
    pltpu.sync_copy(dsts_hbm.at[wid], dst_v)
    pltpu.sync_copy(ones_hbm, ones_v)
    pltpu.sync_copy(zeros_hbm.at[pl.ds(s * _RPT, _RPT)],
                    acc.at[pl.ds(s * _RPT, _RPT)])
    plsc.subcore_barrier()

    def step(j, carry):
        pltpu.sync_copy(ones_v, acc.at[dst_v.at[j]], add=True)
        return carry

    lax.fori_loop(0, _NCH, step, 0)
    plsc.subcore_barrier()
    pltpu.sync_copy(acc.at[pl.ds(s * _RPT, _RPT)],
                    out_hbm.at[c, pl.ds(s * _RPT, _RPT), pl.ds(0, _CP)])


@functools.cache
def _sc_deg():
    return pl.kernel(
        _sc_deg_body,
        out_type=jax.ShapeDtypeStruct((_NC, _NP, 128), jnp.float32),
        mesh=_mesh(),
        compiler_params=_SC_PARAMS,
        scratch_types=[
            pltpu.VMEM((_NCH, _K), jnp.int32),
            pltpu.VMEM((_K, _CP), jnp.float32),
            pltpu.VMEM_SHARED((_NP, _CP), jnp.float32),
        ],
    )


def _sc_scatter_body(g_hbm, srcs_hbm, dsts_hbm, zeros_hbm, out_hbm,
                     src_v, dst_v, bufa, bufb, acc, sga, sgb, ssa, ssb):
    c = lax.axis_index("c")
    s = lax.axis_index("s")
    wid = s * _NC + c
    pltpu.sync_copy(srcs_hbm.at[wid], src_v)
    pltpu.sync_copy(dsts_hbm.at[wid], dst_v)
    pltpu.sync_copy(zeros_hbm.at[pl.ds(s * _RPT, _RPT)],
                    acc.at[pl.ds(s * _RPT, _RPT)])
    plsc.subcore_barrier()

    def gth(j, buf, sem):
        pltpu.async_copy(g_hbm.at[src_v.at[j]], buf, sem)

    def gth_wait(j, buf, sem):
        pltpu.make_async_copy(g_hbm.at[src_v.at[j]], buf, sem).wait()

    def sct(j, buf, sem):
        pltpu.async_copy(buf, acc.at[dst_v.at[j]], sem, add=True)

    def sct_wait(j, buf, sem):
        pltpu.make_async_copy(buf, acc.at[dst_v.at[j]], sem).wait()

    gth(0, bufa, sga)
    gth(1, bufb, sgb)

    def step(i, carry):
        j = 2 * i
        gth_wait(j, bufa, sga)
        sct(j, bufa, ssa)
        gth_wait(j + 1, bufb, sgb)
        sct(j + 1, bufb, ssb)
        sct_wait(j, bufa, ssa)
        gth(j + 2, bufa, sga)
        sct_wait(j + 1, bufb, ssb)
        gth(j + 3, bufb, sgb)
        return carry

    lax.fori_loop(0, (_NCH - 3) // 2, step, 0)
    j = _NCH - 3
    gth_wait(j, bufa, sga)
    sct(j, bufa, ssa)
    gth_wait(j + 1, bufb, sgb)
    sct(j + 1, bufb, ssb)
    sct_wait(j, bufa, ssa)
    gth(j + 2, bufa, sga)
    gth_wait(j + 2, bufa, sga)
    sct(j + 2, bufa, ssa)
    sct_wait(j + 2, bufa, ssa)
    sct_wait(j + 1, bufb, ssb)
    plsc.subcore_barrier()
    pltpu.sync_copy(acc.at[pl.ds(s * _RPT, _RPT)],
                    out_hbm.at[c, pl.ds(s * _RPT, _RPT), pl.ds(0, _H)])


@functools.cache
def _sc_scatter():
    return pl.kernel(
        _sc_scatter_body,
        out_type=jax.ShapeDtypeStruct((_NC, _NP, 128), jnp.float32),
        mesh=_mesh(),
        compiler_params=_SC_PARAMS,
        scratch_types=[
            pltpu.VMEM((_NCH, _K), jnp.int32),
            pltpu.VMEM((_NCH, _K), jnp.int32),
            pltpu.VMEM((_K, _H), jnp.float32),
            pltpu.VMEM((_K, _H), jnp.float32),
            pltpu.VMEM_SHARED((_NP, _H), jnp.float32),
            pltpu.SemaphoreType.DMA,
            pltpu.SemaphoreType.DMA,
            pltpu.SemaphoreType.DMA,
            pltpu.SemaphoreType.DMA,
        ],
    )


def _sc_gather_body(p_hbm, q_hbm, srcs_hbm, dsts_hbm, oz_hbm,
                    src_v, dst_v, bpa, bqa, bpb, bqb,
                    spa, sqa, spb, sqb, swa, swb):
    c = lax.axis_index("c")
    s = lax.axis_index("s")
    wid = s * _NC + c
    base = wid * _EPT
    pltpu.sync_copy(srcs_hbm.at[wid], src_v)
    pltpu.sync_copy(dsts_hbm.at[wid], dst_v)

    def gth(j, bp, bq, sp, sq):
        pltpu.async_copy(p_hbm.at[src_v.at[j]], bp, sp)
        pltpu.async_copy(q_hbm.at[dst_v.at[j]], bq, sq)

    def gth_wait(j, bp, bq, sp, sq):
        pltpu.make_async_copy(p_hbm.at[src_v.at[j]], bp, sp).wait()
        pltpu.make_async_copy(q_hbm.at[dst_v.at[j]], bq, sq).wait()

    def add_rows(bp, bq):
        for r in range(_K):
            bp[r, :] = bp[r, :] + bq[r, :]

    def wb(j, bp, sw):
        pltpu.async_copy(
            bp, oz_hbm.at[pl.ds(base + j * _K, _K), pl.ds(0, _CP)], sw)

    def wb_wait(j, bp, sw):
        pltpu.make_async_copy(
            bp, oz_hbm.at[pl.ds(base + j * _K, _K), pl.ds(0, _CP)], sw).wait()

    gth(0, bpa, bqa, spa, sqa)
    gth(1, bpb, bqb, spb, sqb)

    def step(i, carry):
        j = 2 * i
        gth_wait(j, bpa, bqa, spa, sqa)
        add_rows(bpa, bqa)
        wb(j, bpa, swa)
        gth_wait(j + 1, bpb, bqb, spb, sqb)
        add_rows(bpb, bqb)
        wb(j + 1, bpb, swb)
        wb_wait(j, bpa, swa)
        gth(j + 2, bpa, bqa, spa, sqa)
        wb_wait(j + 1, bpb, swb)
        gth(j + 3, bpb, bqb, spb, sqb)
        return carry

    lax.fori_loop(0, (_NCH - 3) // 2, step, 0)
    j = _NCH - 3
    gth_wait(j, bpa, bqa, spa, sqa)
    add_rows(bpa, bqa)
    wb(j, bpa, swa)
    gth_wait(j + 1, bpb, bqb, spb, sqb)
    add_rows(bpb, bqb)
    wb(j + 1, bpb, swb)
    wb_wait(j, bpa, swa)
    gth(j + 2, bpa, bqa, spa, sqa)
    gth_wait(j + 2, bpa, bqa, spa, sqa)
    add_rows(bpa, bqa)
    wb(j + 2, bpa, swa)
    wb_wait(j + 2, bpa, swa)
    wb_wait(j + 1, bpb, swb)


@functools.cache
def _sc_gather():
    return pl.kernel(
        _sc_gather_body,
        out_type=jax.ShapeDtypeStruct((_E, 128), jnp.float32),
        mesh=_mesh(),
        compiler_params=_SC_PARAMS,
        scratch_types=[
            pltpu.VMEM((_NCH, _K), jnp.int32),
            pltpu.VMEM((_NCH, _K), jnp.int32),
            pltpu.VMEM((_K, _CP), jnp.float32),
            pltpu.VMEM((_K, _CP), jnp.float32),
            pltpu.VMEM((_K, _CP), jnp.float32),
            pltpu.VMEM((_K, _CP), jnp.float32),
            pltpu.SemaphoreType.DMA,
            pltpu.SemaphoreType.DMA,
            pltpu.SemaphoreType.DMA,
            pltpu.SemaphoreType.DMA,
            pltpu.SemaphoreType.DMA,
            pltpu.SemaphoreType.DMA,
        ],
    )


_RB = 2000
_EB = 4000


def _a1_body(deg_ref, x_ref, w_ref, g_ref, dinv_ref):
    deg = (deg_ref[0] + deg_ref[1])[:, :1] + 1.0
    dinv = lax.rsqrt(deg)
    g_ref[...] = jnp.dot(x_ref[...], w_ref[...],
                         preferred_element_type=jnp.float32) * dinv
    dinv_ref[...] = dinv


def _a1(deg, x, w):
    return pl.pallas_call(
        _a1_body,
        grid=(_N // _RB,),
        in_specs=[
            pl.BlockSpec((_NC, _RB, 128), lambda i: (0, i, 0)),
            pl.BlockSpec((_RB, _DIN), lambda i: (i, 0)),
            pl.BlockSpec((_DIN, _H), lambda i: (0, 0)),
        ],
        out_specs=[
            pl.BlockSpec((_RB, _H), lambda i: (i, 0)),
            pl.BlockSpec((_RB, 1), lambda i: (i, 0)),
        ],
        out_shape=[
            jax.ShapeDtypeStruct((_N, _H), jnp.float32),
            jax.ShapeDtypeStruct((_N, 1), jnp.float32),
        ],
    )(deg, x, w)


def _a2_body(g_ref, agg_ref, dinv_ref, w_ref, b_ref, out_ref):
    dinv = dinv_ref[...]
    agg = agg_ref[0][:, :_H] + agg_ref[1][:, :_H]
    h = jnp.maximum(dinv * (g_ref[...] + agg) + b_ref[...], 0.0)
    out_ref[...] = jnp.dot(h, w_ref[...],
                           preferred_element_type=jnp.float32) * dinv


def _a2(g, agg, dinv, w, b):
    return pl.pallas_call(
        _a2_body,
        grid=(_N // _RB,),
        in_specs=[
            pl.BlockSpec((_RB, _H), lambda i: (i, 0)),
            pl.BlockSpec((_NC, _RB, 128), lambda i: (0, i, 0)),
            pl.BlockSpec((_RB, 1), lambda i: (i, 0)),
            pl.BlockSpec((_H, _H), lambda i: (0, 0)),
            pl.BlockSpec((1, _H), lambda i: (0, 0)),
        ],
        out_specs=pl.BlockSpec((_RB, _H), lambda i: (i, 0)),
        out_shape=jax.ShapeDtypeStruct((_N, _H), jnp.float32),
    )(g, agg, dinv, w, b)


def _a4_body(g_ref, agg_ref, dinv_ref, b_ref, wpq_ref, p_ref, q_ref):
    dinv = dinv_ref[...]
    agg = agg_ref[0][:, :_H] + agg_ref[1][:, :_H]
    h = jnp.maximum(dinv * (g_ref[...] + agg) + b_ref[...], 0.0)
    pq = jnp.dot(h, wpq_ref[...], preferred_element_type=jnp.float32)
    p_ref[...] = pq[:, :_CP]
    q_ref[...] = pq[:, _CP:]


def _a4(g, agg, dinv, b, wpq):
    return pl.pallas_call(
        _a4_body,
        grid=(_N // _RB,),
        in_specs=[
            pl.BlockSpec((_RB, _H), lambda i: (i, 0)),
            pl.BlockSpec((_NC, _RB, 128), lambda i: (0, i, 0)),
            pl.BlockSpec((_RB, 1), lambda i: (i, 0)),
            pl.BlockSpec((1, _H), lambda i: (0, 0)),
            pl.BlockSpec((_H, 2 * _CP), lambda i: (0, 0)),
        ],
        out_specs=[
            pl.BlockSpec((_RB, _CP), lambda i: (i, 0)),
            pl.BlockSpec((_RB, _CP), lambda i: (i, 0)),
        ],
        out_shape=[
            jax.ShapeDtypeStruct((_N, _CP), jnp.float32),
            jax.ShapeDtypeStruct((_N, _CP), jnp.float32),
        ],
    )(g, agg, dinv, b, wpq)


def _a5_body(oz_ref, ea_ref, we_ref, b_ref, out_ref):
    z = oz_ref[:, :_CP] + b_ref[...]
    z = z + jnp.dot(ea_ref[...], we_ref[...],
                    preferred_element_type=jnp.float32)
    m = jnp.max(z, axis=1, keepdims=True)
    lse = jnp.log(jnp.sum(jnp.exp(z - m), axis=1, keepdims=True))
    out_ref[...] = (z - m - lse)[:, :_C]


def _a5(oz, ea, we, b):
    return pl.pallas_call(
        _a5_body,
        grid=(_E // _EB,),
        in_specs=[
            pl.BlockSpec((_EB, 128), lambda i: (i, 0)),
            pl.BlockSpec((_EB, 4), lambda i: (i, 0)),
            pl.BlockSpec((4, _CP), lambda i: (0, 0)),
            pl.BlockSpec((1, _CP), lambda i: (0, 0)),
        ],
        out_specs=pl.BlockSpec((_EB, _C), lambda i: (i, 0)),
        out_shape=jax.ShapeDtypeStruct((_E, _C), jnp.float32),
    )(oz, ea, we, b)


def kernel(x, edge_index, edge_attr, W1, b1, W2, b2, W3, b3, Wfc, bfc):
    src = edge_index[0].reshape(_NW, _NCH, _K)
    dst = edge_index[1].reshape(_NW, _NCH, _K)
    zeros_h = jnp.zeros((_NP, _H), jnp.float32)
    zeros_c = jnp.zeros((_NP, _CP), jnp.float32)
    ones_c = jnp.ones((_K, _CP), jnp.float32)

    wp = jnp.pad(Wfc[:_H], ((0, 0), (0, _CP - _C)))
    wq = jnp.pad(Wfc[_H:2 * _H], ((0, 0), (0, _CP - _C)))
    wpq = jnp.concatenate([wp, wq], axis=1)
    we = jnp.pad(Wfc[2 * _H:], ((0, 0), (0, _CP - _C)))
    bpad = jnp.concatenate(
        [bfc, jnp.full((_CP - _C,), -1e30, jnp.float32)]).reshape(1, _CP)

    deg = _sc_deg()(dst, ones_c, zeros_c)
    g1, dinv = _a1(deg, x, W1)
    agg1 = _sc_scatter()(g1, src, dst, zeros_h)
    g2 = _a2(g1, agg1, dinv, W2, b1.reshape(1, _H))
    agg2 = _sc_scatter()(g2, src, dst, zeros_h)
    g3 = _a2(g2, agg2, dinv, W3, b2.reshape(1, _H))
    agg3 = _sc_scatter()(g3, src, dst, zeros_h)
    p, q = _a4(g3, agg3, dinv, b3.reshape(1, _H), wpq)
    oz = _sc_gather()(p, q, src, dst)
    return _a5(oz, edge_attr, we, bpad)
